# baseline (speedup 1.0000x reference)
import jax
import jax.numpy as jnp
from jax.experimental import pallas as pl
from jax.experimental.pallas import tpu as pltpu


def _body(row_ref, col_ref, out_ref, slab_ref):
    h = row_ref.shape[0]
    out_n = row_ref.shape[1]

    @pl.when(pl.program_id(0) == 0)
    def _():
        ident = (
            jax.lax.broadcasted_iota(jnp.int32, (h, h), 0)
            == jax.lax.broadcasted_iota(jnp.int32, (h, h), 1)
        ).astype(jnp.float32)
        dn = (((0,), (0,)), ((), ()))
        row_t = jax.lax.dot_general(row_ref[...], ident, dn,
                                    precision=jax.lax.Precision.HIGHEST)
        col_t = jax.lax.dot_general(col_ref[...], ident, dn,
                                    precision=jax.lax.Precision.HIGHEST)
        top = jnp.broadcast_to(row_t[:, None, :], (out_n, h, h))
        bot = jnp.broadcast_to(col_t[:, :, None], (out_n, h, h))
        slab_ref[...] = jnp.concatenate([top, bot], axis=0)

    out_ref[...] = jnp.broadcast_to(slab_ref[...][None], out_ref.shape)


def kernel(x, row_table, col_table):
    bs, _, h, w = x.shape
    out_n = row_table.shape[1]
    c_total = 2 * out_n
    bblk = 2

    return pl.pallas_call(
        _body,
        grid=(bs // bblk,),
        in_specs=[
            pl.BlockSpec((h, out_n), lambda b: (0, 0)),
            pl.BlockSpec((w, out_n), lambda b: (0, 0)),
        ],
        out_specs=pl.BlockSpec((bblk, c_total, h, w), lambda b: (b, 0, 0, 0)),
        out_shape=jax.ShapeDtypeStruct((bs, c_total, h, w), jnp.float32),
        scratch_shapes=[pltpu.VMEM((c_total, h, w), jnp.float32)],
    )(row_table[:h], col_table[:w])
